# Initial kernel scaffold; baseline (speedup 1.0000x reference)
import jax
import jax.numpy as jnp
from jax.experimental import pallas as pl


def kernel(X, edge_index, W, b):
    raise NotImplementedError("write your pallas kernel here")


# speedup vs baseline: 10.2481x; 10.2481x over previous
import functools

import jax
import jax.numpy as jnp
from jax.experimental import pallas as pl


def _m_kernel(ei_ref, m_ref, *, n_nodes, n_edges):
    src = ei_ref[0:1, :]
    dst = ei_ref[1:2, :]
    iota = jax.lax.broadcasted_iota(jnp.int32, (n_nodes, n_edges), 0)
    s_oh = (iota == src).astype(jnp.float32)
    d_oh = (iota == dst).astype(jnp.float32)
    m_ref[...] = jax.lax.dot_general(
        d_oh, s_oh, (((1,), (1,)), ((), ())),
        preferred_element_type=jnp.float32)


def _x2_kernel(m_ref, x_ref, o_ref):
    o_ref[...] = jax.lax.dot_general(
        m_ref[...], x_ref[...], (((1,), (0,)), ((), ())),
        preferred_element_type=jnp.float32)


def _x1_kernel(m_ref, x_ref, o_ref, *, block_i):
    m = m_ref[...]
    for bb in range(block_i):
        o_ref[bb] = jax.lax.dot_general(
            m, x_ref[bb], (((1,), (0,)), ((), ())),
            preferred_element_type=jnp.float32)


def _mlp_kernel(x_ref, x1_ref, x2_ref, wa_ref, wb_ref, wc_ref, b_ref, o_ref):
    acc = jax.lax.dot_general(
        x_ref[...], wa_ref[...], (((1,), (0,)), ((), ())),
        preferred_element_type=jnp.float32)
    acc += jax.lax.dot_general(
        x1_ref[...], wb_ref[...], (((1,), (0,)), ((), ())),
        preferred_element_type=jnp.float32)
    acc += jax.lax.dot_general(
        x2_ref[...], wc_ref[...], (((1,), (0,)), ((), ())),
        preferred_element_type=jnp.float32)
    o_ref[...] = jnp.maximum(acc + b_ref[...], 0.0)


def kernel(X, edge_index, W, b):
    n, n2, d = X.shape
    assert n == n2
    e = edge_index.shape[1]
    wa, wb, wc = W[:d], W[d:2 * d], W[2 * d:]
    b2 = b.reshape(1, d)

    m = pl.pallas_call(
        functools.partial(_m_kernel, n_nodes=n, n_edges=e),
        out_shape=jax.ShapeDtypeStruct((n, n), jnp.float32),
    )(edge_index)

    x_r = X.reshape(n, n * d)
    cb = (n * d) // 8
    x2_r = pl.pallas_call(
        _x2_kernel,
        grid=(8,),
        in_specs=[
            pl.BlockSpec((n, n), lambda j: (0, 0)),
            pl.BlockSpec((n, cb), lambda j: (0, j)),
        ],
        out_specs=pl.BlockSpec((n, cb), lambda j: (0, j)),
        out_shape=jax.ShapeDtypeStruct((n, n * d), jnp.float32),
    )(m, x_r)

    block_i = 32
    x1 = pl.pallas_call(
        functools.partial(_x1_kernel, block_i=block_i),
        grid=(n // block_i,),
        in_specs=[
            pl.BlockSpec((n, n), lambda i: (0, 0)),
            pl.BlockSpec((block_i, n, d), lambda i: (i, 0, 0)),
        ],
        out_specs=pl.BlockSpec((block_i, n, d), lambda i: (i, 0, 0)),
        out_shape=jax.ShapeDtypeStruct((n, n, d), jnp.float32),
    )(m, X)

    rows = n * n
    rb = rows // 16
    out_flat = pl.pallas_call(
        _mlp_kernel,
        grid=(16,),
        in_specs=[
            pl.BlockSpec((rb, d), lambda r: (r, 0)),
            pl.BlockSpec((rb, d), lambda r: (r, 0)),
            pl.BlockSpec((rb, d), lambda r: (r, 0)),
            pl.BlockSpec((d, d), lambda r: (0, 0)),
            pl.BlockSpec((d, d), lambda r: (0, 0)),
            pl.BlockSpec((d, d), lambda r: (0, 0)),
            pl.BlockSpec((1, d), lambda r: (0, 0)),
        ],
        out_specs=pl.BlockSpec((rb, d), lambda r: (r, 0)),
        out_shape=jax.ShapeDtypeStruct((rows, d), jnp.float32),
    )(X.reshape(rows, d), x1.reshape(rows, d), x2_r.reshape(rows, d),
      wa, wb, wc, b2)

    return out_flat.reshape(n, n, d)


# speedup vs baseline: 16.8683x; 1.6460x over previous
import functools

import jax
import jax.numpy as jnp
from jax.experimental import pallas as pl


def _a_kernel(ei_ref, a_ref, m_ref, *, n_nodes, n_edges):
    src = ei_ref[0:1, :]
    dst = ei_ref[1:2, :]
    iota = jax.lax.broadcasted_iota(jnp.int32, (n_nodes, n_edges), 0)
    s_oh = (iota == src).astype(jnp.float32)
    d_oh = (iota == dst).astype(jnp.float32)
    a_ref[...] = jax.lax.dot_general(
        s_oh, d_oh, (((1,), (1,)), ((), ())),
        preferred_element_type=jnp.float32)
    m_ref[...] = jax.lax.dot_general(
        d_oh, s_oh, (((1,), (1,)), ((), ())),
        preferred_element_type=jnp.float32)


def _y1_kernel(y_ref, a_ref, o_ref):
    o_ref[...] = jax.lax.dot_general(
        y_ref[...], a_ref[...], (((1,), (0,)), ((), ())),
        preferred_element_type=jnp.float32)


def _y2_kernel(m_ref, y_ref, o_ref):
    o_ref[...] = jax.lax.dot_general(
        m_ref[...], y_ref[...], (((1,), (0,)), ((), ())),
        preferred_element_type=jnp.float32)


def _mlp_kernel(y_ref, y1_ref, y2_ref, w_ref, b_ref, o_ref, *, block_i):
    cat = jnp.concatenate(
        [jnp.concatenate([y_ref[i], y1_ref[i], y2_ref[i]], axis=0)
         for i in range(block_i)], axis=1)
    t = jax.lax.dot_general(
        cat, w_ref[...], (((0,), (0,)), ((), ())),
        preferred_element_type=jnp.float32)
    o_ref[...] = jnp.maximum(t + b_ref[...], 0.0)


def kernel(X, edge_index, W, b):
    n, n2, d = X.shape
    assert n == n2
    e = edge_index.shape[1]
    b2 = b.reshape(1, d)

    y = jnp.transpose(X, (0, 2, 1))
    y_f = y.reshape(n * d, n)
    y_r = y.reshape(n, d * n)

    a_mat, m_mat = pl.pallas_call(
        functools.partial(_a_kernel, n_nodes=n, n_edges=e),
        out_shape=(jax.ShapeDtypeStruct((n, n), jnp.float32),
                   jax.ShapeDtypeStruct((n, n), jnp.float32)),
    )(edge_index)

    rb = (n * d) // 8
    y1_f = pl.pallas_call(
        _y1_kernel,
        grid=(8,),
        in_specs=[
            pl.BlockSpec((rb, n), lambda k: (k, 0)),
            pl.BlockSpec((n, n), lambda k: (0, 0)),
        ],
        out_specs=pl.BlockSpec((rb, n), lambda k: (k, 0)),
        out_shape=jax.ShapeDtypeStruct((n * d, n), jnp.float32),
    )(y_f, a_mat)

    cb = (d * n) // 8
    y2_r = pl.pallas_call(
        _y2_kernel,
        grid=(8,),
        in_specs=[
            pl.BlockSpec((n, n), lambda k: (0, 0)),
            pl.BlockSpec((n, cb), lambda k: (0, k)),
        ],
        out_specs=pl.BlockSpec((n, cb), lambda k: (0, k)),
        out_shape=jax.ShapeDtypeStruct((n, d * n), jnp.float32),
    )(m_mat, y_r)

    block_i = 32
    out_flat = pl.pallas_call(
        functools.partial(_mlp_kernel, block_i=block_i),
        grid=(n // block_i,),
        in_specs=[
            pl.BlockSpec((block_i, d, n), lambda k: (k, 0, 0)),
            pl.BlockSpec((block_i, d, n), lambda k: (k, 0, 0)),
            pl.BlockSpec((block_i, d, n), lambda k: (k, 0, 0)),
            pl.BlockSpec((3 * d, d), lambda k: (0, 0)),
            pl.BlockSpec((1, d), lambda k: (0, 0)),
        ],
        out_specs=pl.BlockSpec((block_i * n, d), lambda k: (k, 0)),
        out_shape=jax.ShapeDtypeStruct((n * n, d), jnp.float32),
    )(y, y1_f.reshape(n, d, n), y2_r.reshape(n, d, n), W, b2)

    return out_flat.reshape(n, n, d)


# speedup vs baseline: 20.0879x; 1.1909x over previous
import functools

import jax
import jax.numpy as jnp
from jax.experimental import pallas as pl


def _a_kernel(ei_ref, a_ref, m_ref, *, n_nodes, n_edges):
    src = ei_ref[0:1, :]
    dst = ei_ref[1:2, :]
    iota = jax.lax.broadcasted_iota(jnp.int32, (n_nodes, n_edges), 0)
    s_oh = (iota == src).astype(jnp.float32)
    d_oh = (iota == dst).astype(jnp.float32)
    a_ref[...] = jax.lax.dot_general(
        s_oh, d_oh, (((1,), (1,)), ((), ())),
        preferred_element_type=jnp.float32)
    m_ref[...] = jax.lax.dot_general(
        d_oh, s_oh, (((1,), (1,)), ((), ())),
        preferred_element_type=jnp.float32)


def _y1_kernel(y_ref, a_ref, o_ref, *, block_i):
    n, d = y_ref.shape[2], y_ref.shape[1]
    yf = y_ref[...].reshape(block_i * d, n)
    o_ref[...] = jax.lax.dot_general(
        yf, a_ref[...], (((1,), (0,)), ((), ())),
        preferred_element_type=jnp.float32).reshape(block_i, d, n)


def _y2_kernel(m_ref, y_ref, o_ref, *, block_d):
    m = m_ref[...]
    for dd in range(block_d):
        o_ref[:, dd, :] = jax.lax.dot_general(
            m, y_ref[:, dd, :], (((1,), (0,)), ((), ())),
            preferred_element_type=jnp.float32)


def _mlp_kernel(y_ref, y1_ref, y2_ref, w_ref, b_ref, o_ref, *, block_i):
    cat = jnp.concatenate(
        [jnp.concatenate([y_ref[i], y1_ref[i], y2_ref[i]], axis=0)
         for i in range(block_i)], axis=1)
    t = jax.lax.dot_general(
        cat, w_ref[...], (((0,), (0,)), ((), ())),
        preferred_element_type=jnp.float32)
    o_ref[...] = jnp.maximum(t + b_ref[...], 0.0)


def kernel(X, edge_index, W, b):
    n, n2, d = X.shape
    assert n == n2
    e = edge_index.shape[1]
    b2 = b.reshape(1, d)

    y = jnp.transpose(X, (0, 2, 1))

    a_mat, m_mat = pl.pallas_call(
        functools.partial(_a_kernel, n_nodes=n, n_edges=e),
        out_shape=(jax.ShapeDtypeStruct((n, n), jnp.float32),
                   jax.ShapeDtypeStruct((n, n), jnp.float32)),
    )(edge_index)

    bi1 = n // 8
    y1 = pl.pallas_call(
        functools.partial(_y1_kernel, block_i=bi1),
        grid=(8,),
        in_specs=[
            pl.BlockSpec((bi1, d, n), lambda k: (k, 0, 0)),
            pl.BlockSpec((n, n), lambda k: (0, 0)),
        ],
        out_specs=pl.BlockSpec((bi1, d, n), lambda k: (k, 0, 0)),
        out_shape=jax.ShapeDtypeStruct((n, d, n), jnp.float32),
    )(y, a_mat)

    bd = d // 8
    y2 = pl.pallas_call(
        functools.partial(_y2_kernel, block_d=bd),
        grid=(8,),
        in_specs=[
            pl.BlockSpec((n, n), lambda k: (0, 0)),
            pl.BlockSpec((n, bd, n), lambda k: (0, k, 0)),
        ],
        out_specs=pl.BlockSpec((n, bd, n), lambda k: (0, k, 0)),
        out_shape=jax.ShapeDtypeStruct((n, d, n), jnp.float32),
    )(m_mat, y)

    block_i = 32
    out_flat = pl.pallas_call(
        functools.partial(_mlp_kernel, block_i=block_i),
        grid=(n // block_i,),
        in_specs=[
            pl.BlockSpec((block_i, d, n), lambda k: (k, 0, 0)),
            pl.BlockSpec((block_i, d, n), lambda k: (k, 0, 0)),
            pl.BlockSpec((block_i, d, n), lambda k: (k, 0, 0)),
            pl.BlockSpec((3 * d, d), lambda k: (0, 0)),
            pl.BlockSpec((1, d), lambda k: (0, 0)),
        ],
        out_specs=pl.BlockSpec((block_i * n, d), lambda k: (k, 0)),
        out_shape=jax.ShapeDtypeStruct((n * n, d), jnp.float32),
    )(y, y1, y2, W, b2)

    return out_flat.reshape(n, n, d)
